# baseline (speedup 1.0000x reference)
import functools

import jax
import jax.numpy as jnp
from jax import lax
from jax.experimental import pallas as pl
from jax.experimental.pallas import tpu as pltpu
from jax.experimental.pallas import tpu_sc as plsc

_EMBED_DIM = 1024
_NUM_WORKERS = 32


def kernel(inputs, weights):
    bsz, seq_len = inputs.shape
    out_rows = 2 * seq_len
    row_off = (weights.shape[0] // 2 + 1) - seq_len
    elems = out_rows * _EMBED_DIM
    elems_per_w = elems // _NUM_WORKERS
    elem_off = row_off * _EMBED_DIM

    w_flat = weights.reshape(-1)

    mesh = plsc.VectorSubcoreMesh(core_axis_name="c", subcore_axis_name="s")

    @functools.partial(
        pl.kernel,
        mesh=mesh,
        out_type=jax.ShapeDtypeStruct((elems,), jnp.float32),
    )
    def copy_k(w_hbm, out_hbm):
        wid = lax.axis_index("s") * 2 + lax.axis_index("c")
        base = wid * elems_per_w
        pltpu.sync_copy(
            w_hbm.at[pl.ds(base + elem_off, elems_per_w)],
            out_hbm.at[pl.ds(base, elems_per_w)],
        )

    return copy_k(w_flat).reshape(out_rows, _EMBED_DIM)


# speedup vs baseline: 10.1413x; 10.1413x over previous
import functools

import jax
import jax.numpy as jnp
from jax import lax
from jax.experimental import pallas as pl
from jax.experimental.pallas import tpu as pltpu
from jax.experimental.pallas import tpu_sc as plsc

_EMBED_DIM = 1024
_NUM_WORKERS = 32


def kernel(inputs, weights):
    bsz, seq_len = inputs.shape
    out_rows = 2 * seq_len
    row_off = (weights.shape[0] // 2 + 1) - seq_len
    elems = out_rows * _EMBED_DIM
    elems_per_w = elems // _NUM_WORKERS
    elem_off = row_off * _EMBED_DIM

    w_flat = weights.reshape(-1)

    chunk = 32 * 1024
    nbuf = 3
    nchunks = elems_per_w // chunk

    mesh = plsc.VectorSubcoreMesh(core_axis_name="c", subcore_axis_name="s")

    @functools.partial(
        pl.kernel,
        mesh=mesh,
        out_type=jax.ShapeDtypeStruct((elems,), jnp.float32),
        scratch_types=[pltpu.VMEM((chunk,), jnp.float32)] * nbuf
        + [
            pltpu.SemaphoreType.DMA,
            pltpu.SemaphoreType.DMA,
        ],
    )
    def copy_k(w_hbm, out_hbm, *rest):
        bufs, (sem_in, sem_out) = rest[:nbuf], rest[nbuf:]
        wid = lax.axis_index("s") * 2 + lax.axis_index("c")
        base = wid * elems_per_w

        def in_copy(i):
            return pltpu.make_async_copy(
                w_hbm.at[pl.ds(base + elem_off + i * chunk, chunk)],
                bufs[i % nbuf],
                sem_in,
            )

        def out_copy(i):
            return pltpu.make_async_copy(
                bufs[i % nbuf],
                out_hbm.at[pl.ds(base + i * chunk, chunk)],
                sem_out,
            )

        for j in range(min(nbuf - 1, nchunks)):
            in_copy(j).start()
        for i in range(nchunks):
            j = i + nbuf - 1
            if j < nchunks:
                if j >= nbuf:
                    out_copy(j - nbuf).wait()
                in_copy(j).start()
            in_copy(i).wait()
            out_copy(i).start()
        for i in range(max(0, nchunks - nbuf), nchunks):
            out_copy(i).wait()

    return copy_k(w_flat).reshape(out_rows, _EMBED_DIM)


# speedup vs baseline: 26.0738x; 2.5711x over previous
import functools

import jax
import jax.numpy as jnp
from jax import lax
from jax.experimental import pallas as pl
from jax.experimental.pallas import tpu as pltpu
from jax.experimental.pallas import tpu_sc as plsc

_NUM_WORKERS = 32
_CHUNK_ROWS = 32
_NBUF = 3


def kernel(inputs, weights):
    bsz, seq_len = inputs.shape
    out_rows = 2 * seq_len
    dim = weights.shape[1]
    row_off = (weights.shape[0] // 2 + 1) - seq_len
    rows_per_w = out_rows // _NUM_WORKERS
    nchunks = rows_per_w // _CHUNK_ROWS

    mesh = plsc.VectorSubcoreMesh(core_axis_name="c", subcore_axis_name="s")

    @functools.partial(
        pl.kernel,
        mesh=mesh,
        out_type=jax.ShapeDtypeStruct((out_rows, dim), jnp.float32),
        scratch_types=[pltpu.VMEM((_CHUNK_ROWS, dim), jnp.float32)] * _NBUF
        + [pltpu.VMEM((_CHUNK_ROWS,), jnp.int32)] * _NBUF
        + [
            pltpu.SemaphoreType.DMA,
            pltpu.SemaphoreType.DMA,
        ],
    )
    def copy_k(w_hbm, out_hbm, *rest):
        bufs = rest[:_NBUF]
        idxs = rest[_NBUF : 2 * _NBUF]
        sem_in, sem_out = rest[2 * _NBUF :]
        wid = lax.axis_index("s") * 2 + lax.axis_index("c")
        base = wid * rows_per_w

        def in_copy(i):
            b = i % _NBUF
            start = base + row_off + i * _CHUNK_ROWS
            for k in range(_CHUNK_ROWS // 16):
                idxs[b][pl.ds(k * 16, 16)] = start + k * 16 + lax.iota(
                    jnp.int32, 16
                )
            return pltpu.async_copy(w_hbm.at[idxs[b]], bufs[b], sem_in)

        def out_copy(i):
            return pltpu.make_async_copy(
                bufs[i % _NBUF],
                out_hbm.at[pl.ds(base + i * _CHUNK_ROWS, _CHUNK_ROWS)],
                sem_out,
            )

        pending = []
        for j in range(min(_NBUF - 1, nchunks)):
            pending.append(in_copy(j))
        for i in range(nchunks):
            j = i + _NBUF - 1
            if j < nchunks:
                if j >= _NBUF:
                    out_copy(j - _NBUF).wait()
                pending.append(in_copy(j))
            pending.pop(0).wait()
            out_copy(i).start()
        for i in range(max(0, nchunks - _NBUF), nchunks):
            out_copy(i).wait()

    return copy_k(weights)


# speedup vs baseline: 26.0990x; 1.0010x over previous
import functools

import jax
import jax.numpy as jnp
from jax import lax
from jax.experimental import pallas as pl
from jax.experimental.pallas import tpu as pltpu
from jax.experimental.pallas import tpu_sc as plsc

_NUM_WORKERS = 32
_CHUNK_ROWS = 16
_NBUF = 7


def kernel(inputs, weights):
    bsz, seq_len = inputs.shape
    out_rows = 2 * seq_len
    dim = weights.shape[1]
    row_off = (weights.shape[0] // 2 + 1) - seq_len
    rows_per_w = out_rows // _NUM_WORKERS
    nchunks = rows_per_w // _CHUNK_ROWS

    mesh = plsc.VectorSubcoreMesh(core_axis_name="c", subcore_axis_name="s")

    @functools.partial(
        pl.kernel,
        mesh=mesh,
        out_type=jax.ShapeDtypeStruct((out_rows, dim), jnp.float32),
        scratch_types=[pltpu.VMEM((_CHUNK_ROWS, dim), jnp.float32)] * _NBUF
        + [pltpu.VMEM((_CHUNK_ROWS,), jnp.int32)] * _NBUF
        + [
            pltpu.SemaphoreType.DMA,
            pltpu.SemaphoreType.DMA,
        ],
    )
    def copy_k(w_hbm, out_hbm, *rest):
        bufs = rest[:_NBUF]
        idxs = rest[_NBUF : 2 * _NBUF]
        sem_in, sem_out = rest[2 * _NBUF :]
        wid = lax.axis_index("s") * 2 + lax.axis_index("c")
        base = wid * rows_per_w

        def in_copy(i):
            b = i % _NBUF
            start = base + row_off + i * _CHUNK_ROWS
            for k in range(_CHUNK_ROWS // 16):
                idxs[b][pl.ds(k * 16, 16)] = start + k * 16 + lax.iota(
                    jnp.int32, 16
                )
            return pltpu.async_copy(w_hbm.at[idxs[b]], bufs[b], sem_in)

        def out_copy(i):
            return pltpu.make_async_copy(
                bufs[i % _NBUF],
                out_hbm.at[pl.ds(base + i * _CHUNK_ROWS, _CHUNK_ROWS)],
                sem_out,
            )

        pending = []
        for j in range(min(_NBUF - 1, nchunks)):
            pending.append(in_copy(j))
        for i in range(nchunks):
            j = i + _NBUF - 1
            if j < nchunks:
                if j >= _NBUF:
                    out_copy(j - _NBUF).wait()
                pending.append(in_copy(j))
            pending.pop(0).wait()
            out_copy(i).start()
        for i in range(max(0, nchunks - _NBUF), nchunks):
            out_copy(i).wait()

    return copy_k(weights)


# speedup vs baseline: 26.2128x; 1.0044x over previous
import functools

import jax
import jax.numpy as jnp
from jax import lax
from jax.experimental import pallas as pl
from jax.experimental.pallas import tpu as pltpu
from jax.experimental.pallas import tpu_sc as plsc

_NUM_WORKERS = 32
_CHUNK_ROWS = 16
_NBUF = 7
_LOOKAHEAD = 3


def kernel(inputs, weights):
    bsz, seq_len = inputs.shape
    out_rows = 2 * seq_len
    dim = weights.shape[1]
    row_off = (weights.shape[0] // 2 + 1) - seq_len
    rows_per_w = out_rows // _NUM_WORKERS
    nchunks = rows_per_w // _CHUNK_ROWS

    mesh = plsc.VectorSubcoreMesh(core_axis_name="c", subcore_axis_name="s")

    @functools.partial(
        pl.kernel,
        mesh=mesh,
        out_type=jax.ShapeDtypeStruct((out_rows, dim), jnp.float32),
        scratch_types=[pltpu.VMEM((_CHUNK_ROWS, dim), jnp.float32)] * _NBUF
        + [pltpu.VMEM((_CHUNK_ROWS,), jnp.int32)] * _NBUF
        + [
            pltpu.SemaphoreType.DMA,
            pltpu.SemaphoreType.DMA,
        ],
    )
    def copy_k(w_hbm, out_hbm, *rest):
        bufs = rest[:_NBUF]
        idxs = rest[_NBUF : 2 * _NBUF]
        sem_in, sem_out = rest[2 * _NBUF :]
        wid = lax.axis_index("s") * 2 + lax.axis_index("c")
        base = wid * rows_per_w

        def in_copy(i):
            b = i % _NBUF
            start = base + row_off + i * _CHUNK_ROWS
            for k in range(_CHUNK_ROWS // 16):
                idxs[b][pl.ds(k * 16, 16)] = start + k * 16 + lax.iota(
                    jnp.int32, 16
                )
            return pltpu.async_copy(w_hbm.at[idxs[b]], bufs[b], sem_in)

        def out_copy(i):
            return pltpu.make_async_copy(
                bufs[i % _NBUF],
                out_hbm.at[pl.ds(base + i * _CHUNK_ROWS, _CHUNK_ROWS)],
                sem_out,
            )

        pending = []
        for j in range(min(_NBUF - 1, nchunks)):
            pending.append(in_copy(j))
        for i in range(nchunks):
            j = i + _NBUF - 1
            if j < nchunks:
                if j >= _NBUF:
                    out_copy(j - _NBUF).wait()
                pending.append(in_copy(j))
            pending.pop(0).wait()
            out_copy(i).start()
        for i in range(max(0, nchunks - _NBUF), nchunks):
            out_copy(i).wait()

    return copy_k(weights)
